# Initial kernel scaffold; baseline (speedup 1.0000x reference)
import jax
import jax.numpy as jnp
from jax.experimental import pallas as pl


def kernel(x, expert_mu_kernel, expert_bias, gating_kernel, gating_bias):
    raise NotImplementedError("write your pallas kernel here")


# speedup vs baseline: 2.5089x; 2.5089x over previous
import functools

import jax
import jax.numpy as jnp
from jax.experimental import pallas as pl
from jax.experimental.pallas import tpu as pltpu

N_TOK_ = 8192
D_ = 1024
U_ = 1024
K_ = 8
TILE_N = 1024


def _moe_kernel(x_ref, w_ref, gk_ref, gb_ref, eb_ref, out_ref):
    xf = x_ref[...]
    logits = jax.lax.dot_general(
        xf, gk_ref[...], (((1,), (0,)), ((), ())),
        preferred_element_type=jnp.float32)
    logits = logits + gb_ref[...]
    m = jnp.max(logits, axis=-1, keepdims=True)
    e = jnp.exp(logits - m)
    gates = e / jnp.sum(e, axis=-1, keepdims=True)

    xb = xf.astype(jnp.bfloat16)
    acc = jax.lax.dot_general(
        gates, eb_ref[...], (((1,), (0,)), ((), ())),
        preferred_element_type=jnp.float32)
    for k in range(K_):
        pk = jax.lax.dot_general(
            xb, w_ref[k], (((1,), (0,)), ((), ())),
            preferred_element_type=jnp.float32)
        acc = acc + gates[:, k:k + 1] * pk
    out_ref[...] = acc


@jax.jit
def kernel(x, expert_mu_kernel, expert_bias, gating_kernel, gating_bias):
    w_t = jnp.transpose(expert_mu_kernel, (2, 0, 1)).astype(jnp.bfloat16)
    eb_t = expert_bias.T
    gb = gating_bias.reshape(1, K_)

    grid = (N_TOK_ // TILE_N,)
    return pl.pallas_call(
        _moe_kernel,
        grid=grid,
        in_specs=[
            pl.BlockSpec((TILE_N, D_), lambda i: (i, 0)),
            pl.BlockSpec((K_, D_, U_), lambda i: (0, 0, 0)),
            pl.BlockSpec((D_, K_), lambda i: (0, 0)),
            pl.BlockSpec((1, K_), lambda i: (0, 0)),
            pl.BlockSpec((K_, U_), lambda i: (0, 0)),
        ],
        out_specs=pl.BlockSpec((TILE_N, U_), lambda i: (i, 0)),
        out_shape=jax.ShapeDtypeStruct((N_TOK_, U_), jnp.float32),
        compiler_params=pltpu.CompilerParams(
            dimension_semantics=("arbitrary",),
        ),
    )(x, w_t, gating_kernel, gb, eb_t)
